# Initial kernel scaffold; baseline (speedup 1.0000x reference)
import jax
import jax.numpy as jnp
from jax.experimental import pallas as pl


def kernel(e, x, t, r_ij, neighbors, neighbor_mask, f_ij, W_filter, b_filter, Wq, Wk, Wv, Wo):
    raise NotImplementedError("write your pallas kernel here")


# speedup vs baseline: 1.6240x; 1.6240x over previous
import functools

import jax
import jax.numpy as jnp
from jax import lax
from jax.experimental import pallas as pl
from jax.experimental.pallas import tpu as pltpu
from jax.experimental.pallas import tpu_sc as plsc

A = 10000
NBH = 32
F = 128
G = 32
H = 8
DH = F // H
CUTOFF = 5.0

TA = 200
ROWS = A * NBH
NW = 32
BPW = ROWS // NW
CH = 400


def _h_body(x_ref, e_ref, t_ref, o_ref):
    o_ref[:] = x_ref[:] + e_ref[:] + t_ref[:]


def _compute_h(xs, es, ts):
    return pl.pallas_call(
        _h_body,
        grid=(10,),
        in_specs=[pl.BlockSpec((1000, F), lambda i: (i, 0))] * 3,
        out_specs=pl.BlockSpec((1000, F), lambda i: (i, 0)),
        out_shape=jax.ShapeDtypeStruct((A, F), jnp.float32),
    )(xs, es, ts)


def _sc_gather(h, idx):
    mesh = plsc.VectorSubcoreMesh(core_axis_name="c", subcore_axis_name="s")

    @functools.partial(
        pl.kernel,
        mesh=mesh,
        out_type=jax.ShapeDtypeStruct((ROWS, F), jnp.float32),
        scratch_types=[
            pltpu.VMEM((CH,), jnp.int32),
            pltpu.VMEM((CH, F), jnp.float32),
            pltpu.SemaphoreType.DMA,
        ],
    )
    def gk(h_hbm, idx_hbm, out_hbm, idx_v, rows_v, sem):
        wid = lax.axis_index("s") * 2 + lax.axis_index("c")
        base0 = wid * BPW

        def body(i, carry):
            base = base0 + i * CH
            pltpu.sync_copy(idx_hbm.at[pl.ds(base, CH)], idx_v)
            pltpu.async_copy(h_hbm.at[idx_v], rows_v, sem).wait()
            pltpu.sync_copy(rows_v, out_hbm.at[pl.ds(base, CH)])
            return carry

        lax.fori_loop(0, BPW // CH, body, 0)

    return gk(h, idx)


def _fused_body(nbh_ref, f_ref, r_ref, mk_ref, x_ref, h_ref,
                wf_ref, b_ref, wq_ref, wk_ref, wv_ref, wo_ref, o_ref):
    wfilt = jnp.dot(f_ref[:], wf_ref[:], preferred_element_type=jnp.float32)
    wfilt = wfilt + b_ref[:]
    r = r_ref[:]
    c = 0.5 * (jnp.cos(r * (jnp.pi / CUTOFF)) + 1.0)
    c = jnp.where(r < CUTOFF, c, 0.0) * mk_ref[:]
    m3 = nbh_ref[:] * wfilt.reshape(TA, NBH, F) * c
    m = m3.reshape(TA * NBH, F)
    k = jnp.dot(m, wk_ref[:], preferred_element_type=jnp.float32)
    v = jnp.dot(m, wv_ref[:], preferred_element_type=jnp.float32)
    q = jnp.dot(h_ref[:], wq_ref[:], preferred_element_type=jnp.float32)
    q3 = lax.broadcast_in_dim(q, (TA, NBH, F), (0, 2))
    qk = (q3 * k.reshape(TA, NBH, F)).reshape(TA * NBH, F)
    di = lax.broadcasted_iota(jnp.int32, (F, H), 0) // DH
    hi = lax.broadcasted_iota(jnp.int32, (F, H), 1)
    s = (di == hi).astype(jnp.float32)
    dit = lax.broadcasted_iota(jnp.int32, (H, F), 1) // DH
    hit = lax.broadcasted_iota(jnp.int32, (H, F), 0)
    st = (dit == hit).astype(jnp.float32)
    logits = jnp.dot(qk, s, preferred_element_type=jnp.float32)
    logits = logits.reshape(TA, NBH, H) * (1.0 / (DH ** 0.5))
    mx = jnp.max(logits, axis=1, keepdims=True)
    ex = jnp.exp(logits - mx)
    attn = ex / jnp.sum(ex, axis=1, keepdims=True)
    aw = jnp.dot(attn.reshape(TA * NBH, H), st,
                 preferred_element_type=jnp.float32)
    agg = (aw * v).reshape(TA, NBH, F).sum(axis=1)
    o_ref[:] = x_ref[:] + jnp.dot(agg, wo_ref[:],
                                  preferred_element_type=jnp.float32)


def _fused(nbh, f2, r2, mk2, xs, h, W_filter, b2, Wq, Wk, Wv, Wo):
    grid = (A // TA,)
    return pl.pallas_call(
        _fused_body,
        grid=grid,
        in_specs=[
            pl.BlockSpec((TA, NBH, F), lambda i: (i, 0, 0)),
            pl.BlockSpec((TA * NBH, G), lambda i: (i, 0)),
            pl.BlockSpec((TA, NBH, 1), lambda i: (i, 0, 0)),
            pl.BlockSpec((TA, NBH, 1), lambda i: (i, 0, 0)),
            pl.BlockSpec((TA, F), lambda i: (i, 0)),
            pl.BlockSpec((TA, F), lambda i: (i, 0)),
            pl.BlockSpec((G, F), lambda i: (0, 0)),
            pl.BlockSpec((1, F), lambda i: (0, 0)),
            pl.BlockSpec((F, F), lambda i: (0, 0)),
            pl.BlockSpec((F, F), lambda i: (0, 0)),
            pl.BlockSpec((F, F), lambda i: (0, 0)),
            pl.BlockSpec((F, F), lambda i: (0, 0)),
        ],
        out_specs=pl.BlockSpec((TA, F), lambda i: (i, 0)),
        out_shape=jax.ShapeDtypeStruct((A, F), jnp.float32),
    )(nbh, f2, r2, mk2, xs, h, W_filter, b2, Wq, Wk, Wv, Wo)


def kernel(e, x, t, r_ij, neighbors, neighbor_mask, f_ij,
           W_filter, b_filter, Wq, Wk, Wv, Wo):
    xs, es, ts = x[0], e[0], t[0]
    h = _compute_h(xs, es, ts)
    idx = neighbors[0].reshape(ROWS).astype(jnp.int32)
    nbh = _sc_gather(h, idx).reshape(A, NBH, F)
    f2 = f_ij[0].reshape(ROWS, G)
    r2 = r_ij[0].reshape(A, NBH, 1)
    mk2 = neighbor_mask[0].reshape(A, NBH, 1)
    b2 = b_filter.reshape(1, F)
    out = _fused(nbh, f2, r2, mk2, xs, h, W_filter, b2, Wq, Wk, Wv, Wo)
    return out[None]


# speedup vs baseline: 4.0020x; 2.4642x over previous
import functools

import jax
import jax.numpy as jnp
from jax import lax
from jax.experimental import pallas as pl
from jax.experimental.pallas import tpu as pltpu
from jax.experimental.pallas import tpu_sc as plsc

A = 10000
NBH = 32
F = 128
G = 32
H = 8
DH = F // H
CUTOFF = 5.0

TA = 200
ROWS = A * NBH
NW = 32
BPW = ROWS // NW
CH = 400


def _h_body(x_ref, e_ref, t_ref, o_ref):
    o_ref[:] = x_ref[:] + e_ref[:] + t_ref[:]


def _compute_h(xs, es, ts):
    return pl.pallas_call(
        _h_body,
        grid=(10,),
        in_specs=[pl.BlockSpec((1000, F), lambda i: (i, 0))] * 3,
        out_specs=pl.BlockSpec((1000, F), lambda i: (i, 0)),
        out_shape=jax.ShapeDtypeStruct((A, F), jnp.float32),
    )(xs, es, ts)


def _sc_gather(h, idx):
    mesh = plsc.VectorSubcoreMesh(core_axis_name="c", subcore_axis_name="s")

    @functools.partial(
        pl.kernel,
        mesh=mesh,
        out_type=jax.ShapeDtypeStruct((ROWS, F), jnp.float32),
        scratch_types=[
            pltpu.VMEM((CH,), jnp.int32),
            pltpu.VMEM((CH, F), jnp.float32),
            pltpu.SemaphoreType.DMA,
        ],
    )
    def gk(h_hbm, idx_hbm, out_hbm, idx_v, rows_v, sem):
        wid = lax.axis_index("s") * 2 + lax.axis_index("c")
        base0 = wid * BPW

        def body(i, carry):
            base = base0 + i * CH
            pltpu.sync_copy(idx_hbm.at[pl.ds(base, CH)], idx_v)
            pltpu.async_copy(h_hbm.at[idx_v], rows_v, sem).wait()
            pltpu.sync_copy(rows_v, out_hbm.at[pl.ds(base, CH)])
            return carry

        lax.fori_loop(0, BPW // CH, body, 0)

    return gk(h, idx)


def _fused_body(nbh_ref, f_ref, r_ref, mk_ref, x_ref, h_ref,
                wf_ref, b_ref, wq_ref, wk_ref, wv_ref, wo_ref, o_ref):
    wfilt = jnp.dot(f_ref[:], wf_ref[:], preferred_element_type=jnp.float32)
    wfilt = wfilt + b_ref[:]
    r = r_ref[:]
    c = 0.5 * (jnp.cos(r * (jnp.pi / CUTOFF)) + 1.0)
    c = jnp.where(r < CUTOFF, c, 0.0) * mk_ref[:]
    c3 = lax.broadcast_in_dim(c, (TA, NBH, 1), (0, 1))
    m3 = nbh_ref[:] * wfilt.reshape(TA, NBH, F) * c3
    m = m3.reshape(TA * NBH, F)
    k = jnp.dot(m, wk_ref[:], preferred_element_type=jnp.float32)
    v = jnp.dot(m, wv_ref[:], preferred_element_type=jnp.float32)
    q = jnp.dot(h_ref[:], wq_ref[:], preferred_element_type=jnp.float32)
    q3 = lax.broadcast_in_dim(q, (TA, NBH, F), (0, 2))
    qk = (q3 * k.reshape(TA, NBH, F)).reshape(TA * NBH, F)
    di = lax.broadcasted_iota(jnp.int32, (F, H), 0) // DH
    hi = lax.broadcasted_iota(jnp.int32, (F, H), 1)
    s = (di == hi).astype(jnp.float32)
    dit = lax.broadcasted_iota(jnp.int32, (H, F), 1) // DH
    hit = lax.broadcasted_iota(jnp.int32, (H, F), 0)
    st = (dit == hit).astype(jnp.float32)
    logits = jnp.dot(qk, s, preferred_element_type=jnp.float32)
    logits = logits.reshape(TA, NBH, H) * (1.0 / (DH ** 0.5))
    mx = jnp.max(logits, axis=1, keepdims=True)
    ex = jnp.exp(logits - mx)
    attn = ex / jnp.sum(ex, axis=1, keepdims=True)
    aw = jnp.dot(attn.reshape(TA * NBH, H), st,
                 preferred_element_type=jnp.float32)
    agg = (aw * v).reshape(TA, NBH, F).sum(axis=1)
    o_ref[:] = x_ref[:] + jnp.dot(agg, wo_ref[:],
                                  preferred_element_type=jnp.float32)


def _fused(nbh, f2, r2, mk2, xs, h, W_filter, b2, Wq, Wk, Wv, Wo):
    grid = (A // TA,)
    return pl.pallas_call(
        _fused_body,
        grid=grid,
        in_specs=[
            pl.BlockSpec((TA, NBH, F), lambda i: (i, 0, 0)),
            pl.BlockSpec((TA * NBH, G), lambda i: (i, 0)),
            pl.BlockSpec((TA, NBH), lambda i: (i, 0)),
            pl.BlockSpec((TA, NBH), lambda i: (i, 0)),
            pl.BlockSpec((TA, F), lambda i: (i, 0)),
            pl.BlockSpec((TA, F), lambda i: (i, 0)),
            pl.BlockSpec((G, F), lambda i: (0, 0)),
            pl.BlockSpec((1, F), lambda i: (0, 0)),
            pl.BlockSpec((F, F), lambda i: (0, 0)),
            pl.BlockSpec((F, F), lambda i: (0, 0)),
            pl.BlockSpec((F, F), lambda i: (0, 0)),
            pl.BlockSpec((F, F), lambda i: (0, 0)),
        ],
        out_specs=pl.BlockSpec((TA, F), lambda i: (i, 0)),
        out_shape=jax.ShapeDtypeStruct((A, F), jnp.float32),
    )(nbh, f2, r2, mk2, xs, h, W_filter, b2, Wq, Wk, Wv, Wo)


def kernel(e, x, t, r_ij, neighbors, neighbor_mask, f_ij,
           W_filter, b_filter, Wq, Wk, Wv, Wo):
    xs, es, ts = x[0], e[0], t[0]
    h = _compute_h(xs, es, ts)
    idx = neighbors[0].reshape(ROWS).astype(jnp.int32)
    nbh = _sc_gather(h, idx).reshape(A, NBH, F)
    f2 = f_ij[0].reshape(ROWS, G)
    r2 = r_ij[0]
    mk2 = neighbor_mask[0]
    b2 = b_filter.reshape(1, F)
    out = _fused(nbh, f2, r2, mk2, xs, h, W_filter, b2, Wq, Wk, Wv, Wo)
    return out[None]


# speedup vs baseline: 4.5681x; 1.1415x over previous
import functools

import jax
import jax.numpy as jnp
from jax import lax
from jax.experimental import pallas as pl
from jax.experimental.pallas import tpu as pltpu
from jax.experimental.pallas import tpu_sc as plsc

A = 10000
NBH = 32
F = 128
G = 32
H = 8
DH = F // H
CUTOFF = 5.0

TA = 200
ROWS = A * NBH
NW = 32
P = 5
AP = A // P
ROWS_P = AP * NBH
BPW = ROWS_P // NW
CH = 400
NCH = BPW // CH


def _h_body(x_ref, e_ref, t_ref, o_ref):
    o_ref[:] = x_ref[:] + e_ref[:] + t_ref[:]


def _compute_h(xs, es, ts):
    hb = A // 10
    return pl.pallas_call(
        _h_body,
        grid=(10,),
        in_specs=[pl.BlockSpec((hb, F), lambda i: (i, 0))] * 3,
        out_specs=pl.BlockSpec((hb, F), lambda i: (i, 0)),
        out_shape=jax.ShapeDtypeStruct((A, F), jnp.float32),
    )(xs, es, ts)


def _sc_gather_part(h, idx, p):
    mesh = plsc.VectorSubcoreMesh(core_axis_name="c", subcore_axis_name="s")

    @functools.partial(
        pl.kernel,
        mesh=mesh,
        out_type=jax.ShapeDtypeStruct((ROWS_P, F), jnp.float32),
        scratch_types=[
            pltpu.VMEM((BPW,), jnp.int32),
            pltpu.VMEM((CH, F), jnp.float32),
            pltpu.VMEM((CH, F), jnp.float32),
            pltpu.SemaphoreType.DMA,
            pltpu.SemaphoreType.DMA,
            pltpu.SemaphoreType.DMA,
            pltpu.SemaphoreType.DMA,
        ],
    )
    def gk(h_hbm, idx_hbm, out_hbm, idx_v, r0, r1, g0, g1, s0, s1):
        wid = lax.axis_index("s") * 2 + lax.axis_index("c")
        obase = wid * BPW
        pltpu.sync_copy(idx_hbm.at[pl.ds(p * ROWS_P + obase, BPW)], idx_v)
        rows = [r0, r1]
        gsem = [g0, g1]
        ssem = [s0, s1]
        gcp = [None] * NCH
        scp = [None] * NCH
        gcp[0] = pltpu.async_copy(
            h_hbm.at[idx_v.at[pl.ds(0, CH)]], rows[0], gsem[0])
        for i in range(NCH):
            b = i & 1
            if i + 1 < NCH:
                if i >= 1:
                    scp[i - 1].wait()
                gcp[i + 1] = pltpu.async_copy(
                    h_hbm.at[idx_v.at[pl.ds((i + 1) * CH, CH)]],
                    rows[1 - b], gsem[1 - b])
            gcp[i].wait()
            scp[i] = pltpu.async_copy(
                rows[b], out_hbm.at[pl.ds(obase + i * CH, CH)], ssem[b])
        scp[NCH - 2].wait()
        scp[NCH - 1].wait()

    return gk(h, idx)


def _fused_body(nbh_ref, f_ref, r_ref, mk_ref, x_ref, h_ref,
                wf_ref, b_ref, wq_ref, wk_ref, wv_ref, wo_ref, o_ref):
    wfilt = jnp.dot(f_ref[:], wf_ref[:], preferred_element_type=jnp.float32)
    wfilt = wfilt + b_ref[:]
    r = r_ref[:]
    c = 0.5 * (jnp.cos(r * (jnp.pi / CUTOFF)) + 1.0)
    c = jnp.where(r < CUTOFF, c, 0.0) * mk_ref[:]
    c3 = lax.broadcast_in_dim(c, (TA, NBH, 1), (0, 1))
    m3 = nbh_ref[:] * (wfilt.reshape(TA, NBH, F) * c3)
    m = m3.reshape(TA * NBH, F).astype(jnp.bfloat16)
    k = jnp.dot(m, wk_ref[:], preferred_element_type=jnp.float32)
    v = jnp.dot(m, wv_ref[:], preferred_element_type=jnp.float32)
    q = jnp.dot(h_ref[:].astype(jnp.bfloat16), wq_ref[:],
                preferred_element_type=jnp.float32)
    q3 = lax.broadcast_in_dim(q, (TA, NBH, F), (0, 2))
    qk = (q3 * k.reshape(TA, NBH, F)).reshape(TA * NBH, F)
    di = lax.broadcasted_iota(jnp.int32, (F, H), 0) // DH
    hi = lax.broadcasted_iota(jnp.int32, (F, H), 1)
    s = (di == hi).astype(jnp.bfloat16)
    dit = lax.broadcasted_iota(jnp.int32, (H, F), 1) // DH
    hit = lax.broadcasted_iota(jnp.int32, (H, F), 0)
    st = (dit == hit).astype(jnp.bfloat16)
    logits = jnp.dot(qk.astype(jnp.bfloat16), s,
                     preferred_element_type=jnp.float32)
    logits = logits.reshape(TA, NBH, H) * (1.0 / (DH ** 0.5))
    mx = jnp.max(logits, axis=1, keepdims=True)
    ex = jnp.exp(logits - mx)
    attn = ex / jnp.sum(ex, axis=1, keepdims=True)
    aw = jnp.dot(attn.reshape(TA * NBH, H).astype(jnp.bfloat16), st,
                 preferred_element_type=jnp.float32)
    agg = (aw * v).reshape(TA, NBH, F).sum(axis=1)
    o_ref[:] = x_ref[:] + jnp.dot(agg.astype(jnp.bfloat16), wo_ref[:],
                                  preferred_element_type=jnp.float32)


def _fused_part(p, nbh_p, f2, r2, mk2, xs, h, wfb, b2, wqb, wkb, wvb, wob):
    nb = AP // TA
    o = p * nb
    return pl.pallas_call(
        _fused_body,
        grid=(nb,),
        in_specs=[
            pl.BlockSpec((TA, NBH, F), lambda i: (i, 0, 0)),
            pl.BlockSpec((TA * NBH, G), lambda i: (i + o, 0)),
            pl.BlockSpec((TA, NBH), lambda i: (i + o, 0)),
            pl.BlockSpec((TA, NBH), lambda i: (i + o, 0)),
            pl.BlockSpec((TA, F), lambda i: (i + o, 0)),
            pl.BlockSpec((TA, F), lambda i: (i + o, 0)),
            pl.BlockSpec((G, F), lambda i: (0, 0)),
            pl.BlockSpec((1, F), lambda i: (0, 0)),
            pl.BlockSpec((F, F), lambda i: (0, 0)),
            pl.BlockSpec((F, F), lambda i: (0, 0)),
            pl.BlockSpec((F, F), lambda i: (0, 0)),
            pl.BlockSpec((F, F), lambda i: (0, 0)),
        ],
        out_specs=pl.BlockSpec((TA, F), lambda i: (i, 0)),
        out_shape=jax.ShapeDtypeStruct((AP, F), jnp.float32),
    )(nbh_p, f2, r2, mk2, xs, h, wfb, b2, wqb, wkb, wvb, wob)


def kernel(e, x, t, r_ij, neighbors, neighbor_mask, f_ij,
           W_filter, b_filter, Wq, Wk, Wv, Wo):
    xs, es, ts = x[0], e[0], t[0]
    h = _compute_h(xs, es, ts)
    idx = neighbors[0].reshape(ROWS).astype(jnp.int32)
    f2 = f_ij[0].reshape(ROWS, G).astype(jnp.bfloat16)
    r2 = r_ij[0]
    mk2 = neighbor_mask[0]
    b2 = b_filter.reshape(1, F)
    wfb = W_filter.astype(jnp.bfloat16)
    wqb, wkb = Wq.astype(jnp.bfloat16), Wk.astype(jnp.bfloat16)
    wvb, wob = Wv.astype(jnp.bfloat16), Wo.astype(jnp.bfloat16)
    outs = []
    for p in range(P):
        nbh_p = _sc_gather_part(h, idx, p).reshape(AP, NBH, F)
        outs.append(_fused_part(p, nbh_p, f2, r2, mk2, xs, h,
                                wfb, b2, wqb, wkb, wvb, wob))
    out = jnp.concatenate(outs, axis=0)
    return out[None]


# speedup vs baseline: 4.8748x; 1.0671x over previous
import functools

import jax
import jax.numpy as jnp
from jax import lax
from jax.experimental import pallas as pl
from jax.experimental.pallas import tpu as pltpu
from jax.experimental.pallas import tpu_sc as plsc

A = 10000
NBH = 32
F = 128
G = 32
H = 8
DH = F // H
CUTOFF = 5.0

TA = 200
ROWS = A * NBH
NW = 32
P = 5
AP = A // P
ROWS_P = AP * NBH
BPW = ROWS_P // NW
CH = 400
NCH = BPW // CH
NBUF = 2


def _h_body(x_ref, e_ref, t_ref, o_ref):
    o_ref[:] = x_ref[:] + e_ref[:] + t_ref[:]


def _compute_h(xs, es, ts):
    hb = A // 10
    return pl.pallas_call(
        _h_body,
        grid=(10,),
        in_specs=[pl.BlockSpec((hb, F), lambda i: (i, 0))] * 3,
        out_specs=pl.BlockSpec((hb, F), lambda i: (i, 0)),
        out_shape=jax.ShapeDtypeStruct((A, F), jnp.float32),
    )(xs, es, ts)


def _sc_gather_part(h, idx, p):
    mesh = plsc.VectorSubcoreMesh(core_axis_name="c", subcore_axis_name="s")

    @functools.partial(
        pl.kernel,
        mesh=mesh,
        out_type=jax.ShapeDtypeStruct((ROWS_P, F), jnp.float32),
        scratch_types=[
            pltpu.VMEM((BPW,), jnp.int32),
        ] + [pltpu.VMEM((CH, F), jnp.float32)] * NBUF
          + [pltpu.SemaphoreType.DMA] * (2 * NBUF),
    )
    def gk(h_hbm, idx_hbm, out_hbm, idx_v, *bufs):
        rows = list(bufs[:NBUF])
        gsem = list(bufs[NBUF:2 * NBUF])
        ssem = list(bufs[2 * NBUF:])
        wid = lax.axis_index("s") * 2 + lax.axis_index("c")
        obase = wid * BPW
        pltpu.sync_copy(idx_hbm.at[pl.ds(p * ROWS_P + obase, BPW)], idx_v)
        gcp = [None] * NCH
        scp = [None] * NCH
        for j in range(min(NBUF, NCH)):
            gcp[j] = pltpu.async_copy(
                h_hbm.at[idx_v.at[pl.ds(j * CH, CH)]], rows[j], gsem[j])
        for i in range(NCH):
            b = i % NBUF
            gcp[i].wait()
            scp[i] = pltpu.async_copy(
                rows[b], out_hbm.at[pl.ds(obase + i * CH, CH)], ssem[b])
            nxt = i + NBUF
            if nxt < NCH:
                scp[i].wait()
                gcp[nxt] = pltpu.async_copy(
                    h_hbm.at[idx_v.at[pl.ds(nxt * CH, CH)]], rows[b], gsem[b])
        for i in range(max(0, NCH - NBUF), NCH):
            scp[i].wait()

    return gk(h, idx)


def _fused_body(nbh_ref, f_ref, r_ref, mk_ref, x_ref, h_ref,
                wf_ref, b_ref, wq_ref, wk_ref, wv_ref, wo_ref, o_ref):
    wfilt = jnp.dot(f_ref[:], wf_ref[:], preferred_element_type=jnp.float32)
    wfilt3 = (wfilt + b_ref[:]).reshape(TA, NBH, F)
    r = r_ref[:]
    c = 0.5 * (jnp.cos(r * (jnp.pi / CUTOFF)) + 1.0)
    c = jnp.where(r < CUTOFF, c, 0.0) * mk_ref[:]
    c3 = lax.broadcast_in_dim(c, (TA, NBH, 1), (0, 1))
    m3 = nbh_ref[:] * (wfilt3 * c3)
    m = m3.reshape(TA * NBH, F).astype(jnp.bfloat16)
    k = jnp.dot(m, wk_ref[:], preferred_element_type=jnp.float32)
    v = jnp.dot(m, wv_ref[:], preferred_element_type=jnp.float32)
    q = jnp.dot(h_ref[:].astype(jnp.bfloat16), wq_ref[:],
                preferred_element_type=jnp.float32)
    q3 = lax.broadcast_in_dim(q, (TA, NBH, F), (0, 2))
    qk = (q3 * k.reshape(TA, NBH, F)).reshape(TA * NBH, F)
    di = lax.broadcasted_iota(jnp.int32, (F, H), 0) // DH
    hi = lax.broadcasted_iota(jnp.int32, (F, H), 1)
    s = (di == hi).astype(jnp.bfloat16) * jnp.bfloat16(1.0 / (DH ** 0.5))
    dit = lax.broadcasted_iota(jnp.int32, (H, F), 1) // DH
    hit = lax.broadcasted_iota(jnp.int32, (H, F), 0)
    st = (dit == hit).astype(jnp.bfloat16)
    logits = jnp.dot(qk.astype(jnp.bfloat16), s,
                     preferred_element_type=jnp.float32)
    ex = jnp.exp(logits.reshape(TA, NBH, H))
    den = jnp.sum(ex, axis=1)
    aw = jnp.dot(ex.reshape(TA * NBH, H).astype(jnp.bfloat16), st,
                 preferred_element_type=jnp.float32)
    agg = (aw * v).reshape(TA, NBH, F).sum(axis=1)
    den128 = jnp.dot(den.astype(jnp.bfloat16), st,
                     preferred_element_type=jnp.float32)
    agg = agg / den128
    o_ref[:] = x_ref[:] + jnp.dot(agg.astype(jnp.bfloat16), wo_ref[:],
                                  preferred_element_type=jnp.float32)


def _fused_part(p, nbh_p, f2, r2, mk2, xs, h, wfb, b2, wqb, wkb, wvb, wob):
    nb = AP // TA
    o = p * nb
    return pl.pallas_call(
        _fused_body,
        grid=(nb,),
        in_specs=[
            pl.BlockSpec((TA, NBH, F), lambda i: (i, 0, 0)),
            pl.BlockSpec((TA * NBH, G), lambda i: (i + o, 0)),
            pl.BlockSpec((TA, NBH), lambda i: (i + o, 0)),
            pl.BlockSpec((TA, NBH), lambda i: (i + o, 0)),
            pl.BlockSpec((TA, F), lambda i: (i + o, 0)),
            pl.BlockSpec((TA, F), lambda i: (i + o, 0)),
            pl.BlockSpec((G, F), lambda i: (0, 0)),
            pl.BlockSpec((1, F), lambda i: (0, 0)),
            pl.BlockSpec((F, F), lambda i: (0, 0)),
            pl.BlockSpec((F, F), lambda i: (0, 0)),
            pl.BlockSpec((F, F), lambda i: (0, 0)),
            pl.BlockSpec((F, F), lambda i: (0, 0)),
        ],
        out_specs=pl.BlockSpec((TA, F), lambda i: (i, 0)),
        out_shape=jax.ShapeDtypeStruct((AP, F), jnp.float32),
    )(nbh_p, f2, r2, mk2, xs, h, wfb, b2, wqb, wkb, wvb, wob)


def kernel(e, x, t, r_ij, neighbors, neighbor_mask, f_ij,
           W_filter, b_filter, Wq, Wk, Wv, Wo):
    xs, es, ts = x[0], e[0], t[0]
    h = _compute_h(xs, es, ts)
    idx = neighbors[0].reshape(ROWS).astype(jnp.int32)
    f2 = f_ij[0].reshape(ROWS, G).astype(jnp.bfloat16)
    r2 = r_ij[0]
    mk2 = neighbor_mask[0]
    b2 = b_filter.reshape(1, F)
    wfb = W_filter.astype(jnp.bfloat16)
    wqb, wkb = Wq.astype(jnp.bfloat16), Wk.astype(jnp.bfloat16)
    wvb, wob = Wv.astype(jnp.bfloat16), Wo.astype(jnp.bfloat16)
    outs = []
    for p in range(P):
        nbh_p = _sc_gather_part(h, idx, p).reshape(AP, NBH, F)
        outs.append(_fused_part(p, nbh_p, f2, r2, mk2, xs, h,
                                wfb, b2, wqb, wkb, wvb, wob))
    out = jnp.concatenate(outs, axis=0)
    return out[None]
